# baseline (speedup 1.0000x reference)
import functools

import jax
import jax.numpy as jnp
from jax import lax
from jax.experimental import pallas as pl
from jax.experimental.pallas import tpu as pltpu
from jax.experimental.pallas import tpu_sc as plsc

D = 64
B = 1024
K = 5
NW = 32
BS = B // NW
BN = B * K // NW
BH = BN // 2
SH = BS // 2
L = 16


def _sc_gather_body(w_hbm, src_hbm, pos_hbm, neg_hbm,
                    out_src, out_pp, out_nb,
                    idx_s, idx_p, idx_na, idx_nb,
                    rs, rp, rna, rnb, nb_v, pp_v, sem):
    wid = lax.axis_index("s") * 2 + lax.axis_index("c")
    b0 = wid * BS
    n0 = wid * BN

    pltpu.sync_copy(src_hbm.at[pl.ds(b0, BS)], idx_s)
    pltpu.sync_copy(pos_hbm.at[pl.ds(b0, BS)], idx_p)
    pltpu.sync_copy(neg_hbm.at[pl.ds(n0, BH)], idx_na)
    pltpu.sync_copy(neg_hbm.at[pl.ds(n0 + BH, BH)], idx_nb)

    cs = pltpu.async_copy(w_hbm.at[idx_s], rs, sem)
    cp = pltpu.async_copy(w_hbm.at[idx_p], rp, sem)
    cna = pltpu.async_copy(w_hbm.at[idx_na], rna, sem)
    cnb = pltpu.async_copy(w_hbm.at[idx_nb], rnb, sem)
    cs.wait()
    pltpu.sync_copy(rs, out_src.at[pl.ds(b0, BS)])

    cp.wait()
    for c in range(D // L):
        sl = pl.ds(c * L, L)
        acc = rp[0, sl]
        for r in range(1, BS):
            acc = acc + rp[r, sl]
        pp_v[sl] = acc
    pltpu.sync_copy(pp_v, out_pp.at[wid])

    cna.wait()
    for j in range(SH):
        for c in range(D // L):
            sl = pl.ds(c * L, L)
            acc = rna[K * j, sl]
            for r in range(1, K):
                acc = acc + rna[K * j + r, sl]
            nb_v[j, sl] = acc
    cnb.wait()
    for j in range(SH):
        for c in range(D // L):
            sl = pl.ds(c * L, L)
            acc = rnb[K * j, sl]
            for r in range(1, K):
                acc = acc + rnb[K * j + r, sl]
            nb_v[SH + j, sl] = acc
    pltpu.sync_copy(nb_v, out_nb.at[pl.ds(b0, BS)])


_sc_gather = functools.partial(
    pl.kernel,
    out_type=(
        jax.ShapeDtypeStruct((B, D), jnp.float32),
        jax.ShapeDtypeStruct((NW, D), jnp.float32),
        jax.ShapeDtypeStruct((B, D), jnp.float32),
    ),
    mesh=plsc.VectorSubcoreMesh(core_axis_name="c", subcore_axis_name="s"),
    compiler_params=pltpu.CompilerParams(use_tc_tiling_on_sc=False),
    scratch_types=[
        pltpu.VMEM((BS,), jnp.int32),
        pltpu.VMEM((BS,), jnp.int32),
        pltpu.VMEM((BH,), jnp.int32),
        pltpu.VMEM((BH,), jnp.int32),
        pltpu.VMEM((BS, D), jnp.float32),
        pltpu.VMEM((BS, D), jnp.float32),
        pltpu.VMEM((BH, D), jnp.float32),
        pltpu.VMEM((BH, D), jnp.float32),
        pltpu.VMEM((BS, D), jnp.float32),
        pltpu.VMEM((D,), jnp.float32),
        pltpu.SemaphoreType.DMA,
    ],
)(_sc_gather_body)


def _tc_loss_body(src_ref, pp_ref, nb_ref, out_ref):
    src_rows = src_ref[...]
    s_pos = jnp.sum(pp_ref[...], axis=0, keepdims=True)
    s_src = jnp.sum(src_rows, axis=0, keepdims=True)
    pos_score = jnp.sum(src_rows * s_pos, axis=1, keepdims=True)
    neg_score = jnp.sum(nb_ref[...] * s_src, axis=1, keepdims=True)

    def softplus(z):
        return jnp.maximum(z, 0.0) + jnp.log1p(jnp.exp(-jnp.abs(z)))

    total = jnp.sum(softplus(-pos_score)) + jnp.sum(softplus(neg_score))
    out_ref[0, 0] = total / B


_tc_loss = pl.pallas_call(
    _tc_loss_body,
    out_shape=jax.ShapeDtypeStruct((1, 1), jnp.float32),
    out_specs=pl.BlockSpec(memory_space=pltpu.SMEM),
)


def kernel(src, pos, neg, W):
    src_i = src.astype(jnp.int32)
    pos_i = pos.astype(jnp.int32)
    neg_i = neg.reshape(B * K).astype(jnp.int32)
    src_rows, pos_part, nb = _sc_gather(W, src_i, pos_i, neg_i)
    loss = _tc_loss(src_rows, pos_part, nb)
    return loss[0, 0]
